# baseline (speedup 1.0000x reference)
import functools

import jax
import jax.numpy as jnp
from jax import lax
from jax.experimental import pallas as pl
from jax.experimental.pallas import tpu as pltpu
from jax.experimental.pallas import tpu_sc as plsc

NUM_SPECIAL_TOKENS = 100
EMBED_DIM = 64
SEQ = 200

_NC = 2
_NS = 16
_NW = _NC * _NS

_GSEG = 128
_NBUF = 4

_GROUP_OFFS = tuple(range(0, 192, 16)) + (SEQ - 16,)


def _emb_kernel(n_rows, tok_hbm, word_hbm, spec_hbm, out_hbm,
                tok_v, idx_v, rows_v, spec_v, flags_v, gsem, tsem, osem):
    wid = lax.axis_index("s") * _NC + lax.axis_index("c")
    per_w = n_rows // _NW
    row_w = wid * per_w

    pltpu.sync_copy(spec_hbm, spec_v)

    def tok_copy(g, b):
        return pltpu.make_async_copy(
            tok_hbm.at[row_w + g], tok_v.at[b], tsem.at[b])

    def gather_copies(b):
        return [
            pltpu.make_async_copy(
                word_hbm.at[idx_v.at[b, 0]],
                rows_v.at[b, pl.ds(0, _GSEG)], gsem.at[b]),
            pltpu.make_async_copy(
                word_hbm.at[idx_v.at[b, 1, pl.ds(0, SEQ - _GSEG)]],
                rows_v.at[b, pl.ds(_GSEG, SEQ - _GSEG)], gsem.at[b]),
        ]

    def out_copy(g, b):
        return pltpu.make_async_copy(
            rows_v.at[b], out_hbm.at[row_w + g], osem.at[b])

    def compute_indices(b):
        acc = jnp.zeros((16,), jnp.int32)
        for off in _GROUP_OFFS:
            t = tok_v[b, pl.ds(off, 16)]
            spi = jnp.where(t < NUM_SPECIAL_TOKENS, 1, 0)
            seg, pos = divmod(off, _GSEG)
            idx_v[b, seg, pl.ds(pos, 16)] = jnp.where(
                spi > 0, 0, t + 1 - NUM_SPECIAL_TOKENS)
            acc = acc | spi
        flags_v[b, pl.ds(0, 16)] = acc

    def patch_specials(b):
        acc = flags_v[b, pl.ds(0, 16)]
        row_has_sp = acc[0]
        for l in range(1, 16):
            row_has_sp = row_has_sp | acc[l]

        @pl.when(row_has_sp > 0)
        def _():
            tail_off = _GROUP_OFFS[-1]
            for off in _GROUP_OFFS:
                t = tok_v[b, pl.ds(off, 16)]
                spi = jnp.where(t < NUM_SPECIAL_TOKENS, 1, 0)
                if off == tail_off:
                    covered = tail_off + 16 - 192
                    spi = spi * jnp.where(
                        lax.iota(jnp.int32, 16) >= covered, 1, 0)
                g_has = spi[0]
                for l in range(1, 16):
                    g_has = g_has | spi[l]

                def do_patch(off=off, t=t, spi=spi):
                    idxs = jnp.where(spi > 0, t + 1, 0)
                    row_ids = off + lax.iota(jnp.int32, 16)

                    def col_body(c, _):
                        cvec = jnp.broadcast_to(c, (16,)).astype(jnp.int32)
                        vals = plsc.load_gather(spec_v, [idxs, cvec])
                        cur = plsc.load_gather(rows_v.at[b], [row_ids, cvec])
                        plsc.store_scatter(rows_v.at[b], [row_ids, cvec],
                                           cur + vals)
                        return 0
                    lax.fori_loop(0, EMBED_DIM, col_body, 0)
                pl.when(g_has > 0)(do_patch)

    for g in range(_NBUF):
        tok_copy(g, g).start()
    for g in range(_NBUF - 1):
        tok_copy(g, g).wait()
        compute_indices(g)
        for c in gather_copies(g):
            c.start()

    def outer(i, _):
        for b in range(_NBUF):
            g = i * _NBUF + b
            for c in gather_copies(b):
                c.wait()
            patch_specials(b)
            out_copy(g, b).start()

            @pl.when(g + _NBUF < per_w)
            def _():
                tok_copy(g + _NBUF, b).start()

            b3 = (b + _NBUF - 1) % _NBUF

            @pl.when(g + _NBUF - 1 < per_w)
            def _():
                @pl.when(g >= 1)
                def _():
                    out_copy(g - 1, b3).wait()
                tok_copy(g + _NBUF - 1, b3).wait()
                compute_indices(b3)
                for c in gather_copies(b3):
                    c.start()
        return 0
    lax.fori_loop(0, per_w // _NBUF, outer, 0)

    for g in range(per_w - _NBUF, per_w):
        out_copy(g, g % _NBUF).wait()


@jax.jit
def _emb(tok, word_table, special_table):
    n_rows = tok.shape[0]
    mesh = plsc.VectorSubcoreMesh(core_axis_name="c", subcore_axis_name="s")
    f = functools.partial(
        pl.kernel,
        mesh=mesh,
        compiler_params=pltpu.CompilerParams(
            needs_layout_passes=False, use_tc_tiling_on_sc=False),
        out_type=jax.ShapeDtypeStruct((n_rows, SEQ, EMBED_DIM), jnp.float32),
        scratch_types=[
            pltpu.VMEM((_NBUF, SEQ), jnp.int32),
            pltpu.VMEM((_NBUF, 2, _GSEG), jnp.int32),
            pltpu.VMEM((_NBUF, SEQ, EMBED_DIM), jnp.float32),
            pltpu.VMEM(special_table.shape, jnp.float32),
            pltpu.VMEM((_NBUF, 16), jnp.int32),
            pltpu.SemaphoreType.DMA((_NBUF,)),
            pltpu.SemaphoreType.DMA((_NBUF,)),
            pltpu.SemaphoreType.DMA((_NBUF,)),
        ],
    )(functools.partial(_emb_kernel, n_rows))
    return f(tok, word_table, special_table)


def kernel(token_ids, word_table, special_table):
    return _emb(token_ids.astype(jnp.int32), word_table.astype(jnp.float32),
                special_table.astype(jnp.float32))


# speedup vs baseline: 1.0060x; 1.0060x over previous
import functools

import jax
import jax.numpy as jnp
from jax import lax
from jax.experimental import pallas as pl
from jax.experimental.pallas import tpu as pltpu
from jax.experimental.pallas import tpu_sc as plsc

NUM_SPECIAL_TOKENS = 100
EMBED_DIM = 64
SEQ = 200

_NC = 2
_NS = 16
_NW = _NC * _NS

_GSEG = 128
_NBUF = 4

_GROUP_OFFS = tuple(range(0, 192, 16)) + (SEQ - 16,)


def _emb_kernel(n_rows, toka_hbm, tokb_hbm, word_hbm, spec_hbm, out_hbm,
                tok_v, idx_v, rows_v, spec_v, flags_v, gsem, tsem, osem):
    wid = lax.axis_index("s") * _NC + lax.axis_index("c")
    per_w = n_rows // _NW
    row_w = wid * per_w

    pltpu.sync_copy(spec_hbm, spec_v)

    def tok_copies(g, b):
        return [
            pltpu.make_async_copy(
                toka_hbm.at[row_w + g], tok_v.at[b, pl.ds(0, 128)],
                tsem.at[b]),
            pltpu.make_async_copy(
                tokb_hbm.at[row_w + g], tok_v.at[b, pl.ds(128, SEQ - 128)],
                tsem.at[b]),
        ]

    def gather_copies(b):
        return [
            pltpu.make_async_copy(
                word_hbm.at[idx_v.at[b, 0]],
                rows_v.at[b, pl.ds(0, _GSEG)], gsem.at[b]),
            pltpu.make_async_copy(
                word_hbm.at[idx_v.at[b, 1, pl.ds(0, SEQ - _GSEG)]],
                rows_v.at[b, pl.ds(_GSEG, SEQ - _GSEG)], gsem.at[b]),
        ]

    def out_copy(g, b):
        return pltpu.make_async_copy(
            rows_v.at[b], out_hbm.at[row_w + g], osem.at[b])

    def compute_indices(b):
        acc = jnp.zeros((16,), jnp.int32)
        for off in _GROUP_OFFS:
            t = tok_v[b, pl.ds(off, 16)]
            spi = jnp.where(t < NUM_SPECIAL_TOKENS, 1, 0)
            seg, pos = divmod(off, _GSEG)
            idx_v[b, seg, pl.ds(pos, 16)] = jnp.where(
                spi > 0, 0, t + 1 - NUM_SPECIAL_TOKENS)
            acc = acc | spi
        flags_v[b, pl.ds(0, 16)] = acc

    def patch_specials(b):
        acc = flags_v[b, pl.ds(0, 16)]
        row_has_sp = acc[0]
        for l in range(1, 16):
            row_has_sp = row_has_sp | acc[l]

        @pl.when(row_has_sp > 0)
        def _():
            tail_off = _GROUP_OFFS[-1]
            for off in _GROUP_OFFS:
                t = tok_v[b, pl.ds(off, 16)]
                spi = jnp.where(t < NUM_SPECIAL_TOKENS, 1, 0)
                if off == tail_off:
                    covered = tail_off + 16 - 192
                    spi = spi * jnp.where(
                        lax.iota(jnp.int32, 16) >= covered, 1, 0)
                g_has = spi[0]
                for l in range(1, 16):
                    g_has = g_has | spi[l]

                def do_patch(off=off, t=t, spi=spi):
                    idxs = jnp.where(spi > 0, t + 1, 0)
                    row_ids = off + lax.iota(jnp.int32, 16)

                    def col_body(c, _):
                        cvec = jnp.broadcast_to(c, (16,)).astype(jnp.int32)
                        vals = plsc.load_gather(spec_v, [idxs, cvec])
                        cur = plsc.load_gather(rows_v.at[b], [row_ids, cvec])
                        plsc.store_scatter(rows_v.at[b], [row_ids, cvec],
                                           cur + vals)
                        return 0
                    lax.fori_loop(0, EMBED_DIM, col_body, 0)
                pl.when(g_has > 0)(do_patch)

    for g in range(_NBUF):
        for c in tok_copies(g, g):
            c.start()
    for g in range(_NBUF - 1):
        for c in tok_copies(g, g):
            c.wait()
        compute_indices(g)
        for c in gather_copies(g):
            c.start()

    def outer(i, _):
        for b in range(_NBUF):
            g = i * _NBUF + b
            for c in gather_copies(b):
                c.wait()
            patch_specials(b)
            out_copy(g, b).start()

            @pl.when(g + _NBUF < per_w)
            def _():
                for c in tok_copies(g + _NBUF, b):
                    c.start()

            b3 = (b + _NBUF - 1) % _NBUF

            @pl.when(g + _NBUF - 1 < per_w)
            def _():
                @pl.when(g >= 1)
                def _():
                    out_copy(g - 1, b3).wait()
                for c in tok_copies(g + _NBUF - 1, b3):
                    c.wait()
                compute_indices(b3)
                for c in gather_copies(b3):
                    c.start()
        return 0
    lax.fori_loop(0, per_w // _NBUF, outer, 0)

    for g in range(per_w - _NBUF, per_w):
        out_copy(g, g % _NBUF).wait()


@jax.jit
def _emb(tok, word_table, special_table):
    n_rows = tok.shape[0]
    toka = tok[:, :128]
    tokb = tok[:, 128:]
    mesh = plsc.VectorSubcoreMesh(core_axis_name="c", subcore_axis_name="s")
    f = functools.partial(
        pl.kernel,
        mesh=mesh,
        compiler_params=pltpu.CompilerParams(
            needs_layout_passes=False, use_tc_tiling_on_sc=False),
        out_type=jax.ShapeDtypeStruct((n_rows, SEQ, EMBED_DIM), jnp.float32),
        scratch_types=[
            pltpu.VMEM((_NBUF, SEQ), jnp.int32),
            pltpu.VMEM((_NBUF, 2, _GSEG), jnp.int32),
            pltpu.VMEM((_NBUF, SEQ, EMBED_DIM), jnp.float32),
            pltpu.VMEM(special_table.shape, jnp.float32),
            pltpu.VMEM((_NBUF, 16), jnp.int32),
            pltpu.SemaphoreType.DMA((_NBUF,)),
            pltpu.SemaphoreType.DMA((_NBUF,)),
            pltpu.SemaphoreType.DMA((_NBUF,)),
        ],
    )(functools.partial(_emb_kernel, n_rows))
    return f(toka, tokb, word_table, special_table)


def kernel(token_ids, word_table, special_table):
    return _emb(token_ids.astype(jnp.int32), word_table.astype(jnp.float32),
                special_table.astype(jnp.float32))
